# baseline (device time: 110893 ns/iter reference)
import jax
import jax.numpy as jnp
from jax import lax
from jax.experimental import pallas as pl
from jax.experimental.pallas import tpu as pltpu


def kernel(x, W, labels):
    T, D = x.shape
    _, V = W.shape
    BV = 2048
    nblk = V // BV

    def body(x_ref, w_ref, l_ref, out_ref, lg_ref, acc_ref, recv_ref,
             send_sem, recv_sem):
        j = pl.program_id(0)
        my_x = lax.axis_index("x")
        my_y = lax.axis_index("y")
        my_z = lax.axis_index("z")

        @pl.when(j < nblk)
        def _():
            lg_ref[j % 2] = jnp.dot(x_ref[:, :], w_ref[:, :],
                                    preferred_element_type=jnp.float32)

        @pl.when(j > 0)
        def _():
            lg = lg_ref[(j - 1) % 2]
            bs = jnp.sum(jnp.exp(lg), axis=1)
            offset = my_x * V + (j - 1) * BV
            loc = l_ref[:] - offset
            cols = lax.broadcasted_iota(jnp.int32, (T, BV), 1)
            lab = jnp.sum(jnp.where(cols == loc[:, None], lg, 0.0), axis=1)

            @pl.when(j == 1)
            def _():
                acc_ref[0, :] = bs
                acc_ref[1, :] = lab

            @pl.when(j > 1)
            def _():
                acc_ref[0, :] = acc_ref[0, :] + bs
                acc_ref[1, :] = acc_ref[1, :] + lab

        @pl.when(j == nblk)
        def _():
            partner = (1 - my_x, my_y, my_z)
            barrier = pltpu.get_barrier_semaphore()
            pl.semaphore_signal(barrier, inc=1, device_id=partner,
                                device_id_type=pl.DeviceIdType.MESH)
            pl.semaphore_wait(barrier, 1)

            rdma = pltpu.make_async_remote_copy(
                src_ref=acc_ref,
                dst_ref=recv_ref,
                send_sem=send_sem,
                recv_sem=recv_sem,
                device_id=partner,
                device_id_type=pl.DeviceIdType.MESH,
            )
            rdma.start()
            rdma.wait()

            s = acc_ref[0, :] + recv_ref[0, :]
            g = acc_ref[1, :] + recv_ref[1, :]
            out_ref[:] = jnp.log(s) - g

    return pl.pallas_call(
        body,
        grid=(nblk + 1,),
        out_shape=jax.ShapeDtypeStruct((T,), jnp.float32),
        in_specs=[
            pl.BlockSpec((T, D), lambda j: (0, 0)),
            pl.BlockSpec((D, BV), lambda j: (0, jnp.minimum(j, nblk - 1))),
            pl.BlockSpec((T,), lambda j: (0,)),
        ],
        out_specs=pl.BlockSpec((T,), lambda j: (0,)),
        scratch_shapes=[
            pltpu.VMEM((2, T, BV), jnp.float32),
            pltpu.VMEM((2, T), jnp.float32),
            pltpu.VMEM((2, T), jnp.float32),
            pltpu.SemaphoreType.DMA,
            pltpu.SemaphoreType.DMA,
        ],
        compiler_params=pltpu.CompilerParams(
            dimension_semantics=("arbitrary",),
            collective_id=0,
            vmem_limit_bytes=100 * 1024 * 1024,
        ),
    )(x, W, labels)
